# Initial kernel scaffold; baseline (speedup 1.0000x reference)
import jax
import jax.numpy as jnp
from jax.experimental import pallas as pl


def kernel(a, priors, W, b, gamma, beta):
    raise NotImplementedError("write your pallas kernel here")


# speedup vs baseline: 8.0211x; 8.0211x over previous
import jax
import jax.numpy as jnp
from jax.experimental import pallas as pl

_VBS = 1024
_EPS = 1e-5
_N_BISECT = 22


def _fused_chunk(a_ref, p_ref, wt_ref, b_ref, g_ref, bt_ref, o_ref):
    h = jnp.dot(a_ref[...], wt_ref[...], preferred_element_type=jnp.float32)
    h = h + b_ref[...]
    mean = jnp.mean(h, axis=0, keepdims=True)
    var = jnp.mean(jnp.square(h - mean), axis=0, keepdims=True)
    h = (h - mean) * jax.lax.rsqrt(var + _EPS) * g_ref[...] + bt_ref[...]
    z = h * p_ref[...]
    z = z - jnp.max(z, axis=-1, keepdims=True)
    lo = jnp.full(z.shape[:-1] + (1,), -1.0, dtype=jnp.float32)
    hi = jnp.zeros(z.shape[:-1] + (1,), dtype=jnp.float32)
    for _ in range(_N_BISECT):
        mid = 0.5 * (lo + hi)
        f = jnp.sum(jnp.maximum(z - mid, 0.0), axis=-1, keepdims=True)
        pred = f > 1.0
        lo = jnp.where(pred, mid, lo)
        hi = jnp.where(pred, hi, mid)
    supp = (z > lo).astype(jnp.float32)
    k = jnp.sum(supp, axis=-1, keepdims=True)
    s = jnp.sum(z * supp, axis=-1, keepdims=True)
    tau = (s - 1.0) / k
    o_ref[...] = jnp.maximum(z - tau, 0.0)


def kernel(a, priors, W, b, gamma, beta):
    n, d_a = a.shape
    inp_dim = W.shape[0]
    chunk = _VBS if n > _VBS else n
    wt = W.T
    b2 = b.reshape(1, inp_dim)
    g2 = gamma.reshape(1, inp_dim)
    bt2 = beta.reshape(1, inp_dim)
    return pl.pallas_call(
        _fused_chunk,
        grid=(n // chunk,),
        in_specs=[
            pl.BlockSpec((chunk, d_a), lambda i: (i, 0)),
            pl.BlockSpec((chunk, inp_dim), lambda i: (i, 0)),
            pl.BlockSpec((d_a, inp_dim), lambda i: (0, 0)),
            pl.BlockSpec((1, inp_dim), lambda i: (0, 0)),
            pl.BlockSpec((1, inp_dim), lambda i: (0, 0)),
            pl.BlockSpec((1, inp_dim), lambda i: (0, 0)),
        ],
        out_specs=pl.BlockSpec((chunk, inp_dim), lambda i: (i, 0)),
        out_shape=jax.ShapeDtypeStruct((n, inp_dim), jnp.float32),
    )(a, priors, wt, b2, g2, bt2)


# speedup vs baseline: 12.9559x; 1.6152x over previous
import jax
import jax.numpy as jnp
from jax.experimental import pallas as pl

_VBS = 1024
_EPS = 1e-5
_N_BISECT = 16


def _fused_chunk(a_ref, p_ref, wt_ref, b_ref, g_ref, bt_ref, o_ref):
    h = jnp.dot(a_ref[...], wt_ref[...], preferred_element_type=jnp.float32)
    h = h + b_ref[...]
    mean = jnp.mean(h, axis=0, keepdims=True)
    var = jnp.mean(jnp.square(h - mean), axis=0, keepdims=True)
    h = (h - mean) * jax.lax.rsqrt(var + _EPS) * g_ref[...] + bt_ref[...]
    z = h * p_ref[...]
    zt = z.T
    m = jnp.max(zt, axis=0, keepdims=True)
    zt = zt - m
    lo = jnp.full((1, zt.shape[1]), -1.0, dtype=jnp.float32)
    hi = jnp.zeros((1, zt.shape[1]), dtype=jnp.float32)
    for _ in range(_N_BISECT):
        mid = 0.5 * (lo + hi)
        f = jnp.sum(jnp.maximum(zt - mid, 0.0), axis=0, keepdims=True)
        pred = f > 1.0
        lo = jnp.where(pred, mid, lo)
        hi = jnp.where(pred, hi, mid)
    tau = lo
    for _ in range(2):
        supp = (zt > tau).astype(jnp.float32)
        k = jnp.sum(supp, axis=0, keepdims=True)
        s = jnp.sum(zt * supp, axis=0, keepdims=True)
        tau = (s - 1.0) / k
    shift = (m + tau).T
    o_ref[...] = jnp.maximum(z - shift, 0.0)


def kernel(a, priors, W, b, gamma, beta):
    n, d_a = a.shape
    inp_dim = W.shape[0]
    chunk = _VBS if n > _VBS else n
    wt = W.T
    b2 = b.reshape(1, inp_dim)
    g2 = gamma.reshape(1, inp_dim)
    bt2 = beta.reshape(1, inp_dim)
    return pl.pallas_call(
        _fused_chunk,
        grid=(n // chunk,),
        in_specs=[
            pl.BlockSpec((chunk, d_a), lambda i: (i, 0)),
            pl.BlockSpec((chunk, inp_dim), lambda i: (i, 0)),
            pl.BlockSpec((d_a, inp_dim), lambda i: (0, 0)),
            pl.BlockSpec((1, inp_dim), lambda i: (0, 0)),
            pl.BlockSpec((1, inp_dim), lambda i: (0, 0)),
            pl.BlockSpec((1, inp_dim), lambda i: (0, 0)),
        ],
        out_specs=pl.BlockSpec((chunk, inp_dim), lambda i: (i, 0)),
        out_shape=jax.ShapeDtypeStruct((n, inp_dim), jnp.float32),
    )(a, priors, wt, b2, g2, bt2)


# speedup vs baseline: 14.2216x; 1.0977x over previous
import jax
import jax.numpy as jnp
from jax.experimental import pallas as pl
from jax.experimental.pallas import tpu as pltpu

_VBS = 1024
_EPS = 1e-5
_N_BISECT = 12


def _fused_chunk(a_ref, p_ref, wt_ref, b_ref, g_ref, bt_ref, o_ref):
    h = jnp.dot(a_ref[...], wt_ref[...], preferred_element_type=jnp.float32)
    h = h + b_ref[...]
    mean = jnp.mean(h, axis=0, keepdims=True)
    var = jnp.mean(jnp.square(h - mean), axis=0, keepdims=True)
    h = (h - mean) * jax.lax.rsqrt(var + _EPS) * g_ref[...] + bt_ref[...]
    z = h * p_ref[...]
    zt = z.T
    m = jnp.max(zt, axis=0, keepdims=True)
    zt = zt - m
    lo = jnp.full((1, zt.shape[1]), -1.0, dtype=jnp.float32)
    hi = jnp.zeros((1, zt.shape[1]), dtype=jnp.float32)
    for _ in range(_N_BISECT):
        mid = 0.5 * (lo + hi)
        f = jnp.sum(jnp.maximum(zt - mid, 0.0), axis=0, keepdims=True)
        pred = f > 1.0
        lo = jnp.where(pred, mid, lo)
        hi = jnp.where(pred, hi, mid)
    tau = lo
    for _ in range(2):
        supp = (zt > tau).astype(jnp.float32)
        k = jnp.sum(supp, axis=0, keepdims=True)
        s = jnp.sum(zt * supp, axis=0, keepdims=True)
        tau = (s - 1.0) / k
    shift = (m + tau).T
    o_ref[...] = jnp.maximum(z - shift, 0.0)


def kernel(a, priors, W, b, gamma, beta):
    n, d_a = a.shape
    inp_dim = W.shape[0]
    chunk = _VBS if n > _VBS else n
    wt = W.T
    b2 = b.reshape(1, inp_dim)
    g2 = gamma.reshape(1, inp_dim)
    bt2 = beta.reshape(1, inp_dim)
    return pl.pallas_call(
        _fused_chunk,
        grid=(n // chunk,),
        in_specs=[
            pl.BlockSpec((chunk, d_a), lambda i: (i, 0)),
            pl.BlockSpec((chunk, inp_dim), lambda i: (i, 0)),
            pl.BlockSpec((d_a, inp_dim), lambda i: (0, 0)),
            pl.BlockSpec((1, inp_dim), lambda i: (0, 0)),
            pl.BlockSpec((1, inp_dim), lambda i: (0, 0)),
            pl.BlockSpec((1, inp_dim), lambda i: (0, 0)),
        ],
        out_specs=pl.BlockSpec((chunk, inp_dim), lambda i: (i, 0)),
        out_shape=jax.ShapeDtypeStruct((n, inp_dim), jnp.float32),
        compiler_params=pltpu.CompilerParams(
            dimension_semantics=("parallel",),
        ),
    )(a, priors, wt, b2, g2, bt2)


# speedup vs baseline: 18.0115x; 1.2665x over previous
import jax
import jax.numpy as jnp
from jax.experimental import pallas as pl
from jax.experimental.pallas import tpu as pltpu

_VBS = 1024
_EPS = 1e-5
_N_BISECT = 12
_N_NEWTON = 2


def _fused_chunk(a_ref, p_ref, wt_ref, b_ref, g_ref, bt_ref, o_ref):
    d = jnp.float32(o_ref.shape[1])
    h = jnp.dot(a_ref[...], wt_ref[...], preferred_element_type=jnp.float32)
    h = h + b_ref[...]
    outs = []
    for j in range(0, h.shape[0], _VBS):
        hc = h[j:j + _VBS]
        mean = jnp.mean(hc, axis=0, keepdims=True)
        var = jnp.mean(jnp.square(hc - mean), axis=0, keepdims=True)
        outs.append((hc - mean) * jax.lax.rsqrt(var + _EPS))
    hn = outs[0] if len(outs) == 1 else jnp.concatenate(outs, axis=0)
    z = (hn * g_ref[...] + bt_ref[...]) * p_ref[...]
    zt = z.T
    m = jnp.max(zt, axis=0, keepdims=True)
    lo = m - 1.0
    hi = m
    for _ in range(_N_BISECT):
        mid = 0.5 * (lo + hi)
        fs = jnp.sum(jnp.maximum(zt, mid), axis=0, keepdims=True)
        pred = fs - d * mid > 1.0
        lo = jnp.where(pred, mid, lo)
        hi = jnp.where(pred, hi, mid)
    tau = lo
    for _ in range(_N_NEWTON):
        fs = jnp.sum(jnp.maximum(zt, tau), axis=0, keepdims=True)
        k = jnp.sum((zt > tau).astype(jnp.float32), axis=0, keepdims=True)
        tau = tau + (fs - d * tau - 1.0) / k
    o_ref[...] = jnp.maximum(z - tau.T, 0.0)


def kernel(a, priors, W, b, gamma, beta):
    n, d_a = a.shape
    inp_dim = W.shape[0]
    block = min(n, 2 * _VBS)
    wt = W.T
    b2 = b.reshape(1, inp_dim)
    g2 = gamma.reshape(1, inp_dim)
    bt2 = beta.reshape(1, inp_dim)
    return pl.pallas_call(
        _fused_chunk,
        grid=(n // block,),
        in_specs=[
            pl.BlockSpec((block, d_a), lambda i: (i, 0)),
            pl.BlockSpec((block, inp_dim), lambda i: (i, 0)),
            pl.BlockSpec((d_a, inp_dim), lambda i: (0, 0)),
            pl.BlockSpec((1, inp_dim), lambda i: (0, 0)),
            pl.BlockSpec((1, inp_dim), lambda i: (0, 0)),
            pl.BlockSpec((1, inp_dim), lambda i: (0, 0)),
        ],
        out_specs=pl.BlockSpec((block, inp_dim), lambda i: (i, 0)),
        out_shape=jax.ShapeDtypeStruct((n, inp_dim), jnp.float32),
        compiler_params=pltpu.CompilerParams(
            dimension_semantics=("parallel",),
        ),
    )(a, priors, wt, b2, g2, bt2)


# speedup vs baseline: 18.7793x; 1.0426x over previous
import jax
import jax.numpy as jnp
from jax.experimental import pallas as pl
from jax.experimental.pallas import tpu as pltpu

_VBS = 1024
_EPS = 1e-5
_N_BISECT = 12
_N_NEWTON = 2


def _fused_chunk(a_ref, p_ref, wt_ref, b_ref, g_ref, bt_ref, o_ref):
    d = jnp.float32(o_ref.shape[1])
    h = jnp.dot(a_ref[...], wt_ref[...], preferred_element_type=jnp.float32)
    h = h + b_ref[...]
    outs = []
    for j in range(0, h.shape[0], _VBS):
        hc = h[j:j + _VBS]
        mean = jnp.mean(hc, axis=0, keepdims=True)
        var = jnp.mean(jnp.square(hc - mean), axis=0, keepdims=True)
        outs.append((hc - mean) * jax.lax.rsqrt(var + _EPS))
    hn = outs[0] if len(outs) == 1 else jnp.concatenate(outs, axis=0)
    z = (hn * g_ref[...] + bt_ref[...]) * p_ref[...]
    zt = z.T
    m = jnp.max(zt, axis=0, keepdims=True)
    lo = m - 1.0
    hi = m
    for _ in range(_N_BISECT):
        mid = 0.5 * (lo + hi)
        fs = jnp.sum(jnp.maximum(zt, mid), axis=0, keepdims=True)
        pred = fs - d * mid > 1.0
        lo = jnp.where(pred, mid, lo)
        hi = jnp.where(pred, hi, mid)
    tau = lo
    for _ in range(_N_NEWTON):
        fs = jnp.sum(jnp.maximum(zt, tau), axis=0, keepdims=True)
        k = jnp.sum((zt > tau).astype(jnp.float32), axis=0, keepdims=True)
        tau = tau + (fs - d * tau - 1.0) / k
    o_ref[...] = jnp.maximum(z - tau.T, 0.0)


def kernel(a, priors, W, b, gamma, beta):
    n, d_a = a.shape
    inp_dim = W.shape[0]
    block = min(n, 4 * _VBS)
    wt = W.T
    b2 = b.reshape(1, inp_dim)
    g2 = gamma.reshape(1, inp_dim)
    bt2 = beta.reshape(1, inp_dim)
    return pl.pallas_call(
        _fused_chunk,
        grid=(n // block,),
        in_specs=[
            pl.BlockSpec((block, d_a), lambda i: (i, 0)),
            pl.BlockSpec((block, inp_dim), lambda i: (i, 0)),
            pl.BlockSpec((d_a, inp_dim), lambda i: (0, 0)),
            pl.BlockSpec((1, inp_dim), lambda i: (0, 0)),
            pl.BlockSpec((1, inp_dim), lambda i: (0, 0)),
            pl.BlockSpec((1, inp_dim), lambda i: (0, 0)),
        ],
        out_specs=pl.BlockSpec((block, inp_dim), lambda i: (i, 0)),
        out_shape=jax.ShapeDtypeStruct((n, inp_dim), jnp.float32),
        compiler_params=pltpu.CompilerParams(
            dimension_semantics=("parallel",),
        ),
    )(a, priors, wt, b2, g2, bt2)
